# baseline (speedup 1.0000x reference)
import functools

import jax
import jax.numpy as jnp
import numpy as np
from jax import lax
from jax.experimental import pallas as pl
from jax.experimental.pallas import tpu as pltpu

B = 128
V = 100000
CCH = 8192
NCH = -(-V // CCH)
INTMAX = 2**31 - 1

K0 = 0
K1 = 42
KS2 = K0 ^ K1 ^ 0x1BD11BDA
U32 = jnp.uint32

_ROT1 = (13, 15, 26, 6)
_ROT2 = (17, 29, 16, 24)


def _rounds(x0, x1, rots):
    for r in rots:
        x0 = x0 + x1
        x1 = (x1 << U32(r)) | (x1 >> U32(32 - r))
        x1 = x1 ^ x0
    return x0, x1


def _threefry_bits(ctr):
    x0 = jnp.zeros_like(ctr) + U32(K0)
    x1 = ctr + U32(K1)
    x0, x1 = _rounds(x0, x1, _ROT1)
    x0 = x0 + U32(K1)
    x1 = x1 + U32(KS2 + 1)
    x0, x1 = _rounds(x0, x1, _ROT2)
    x0 = x0 + U32(KS2)
    x1 = x1 + U32(K0 + 2)
    x0, x1 = _rounds(x0, x1, _ROT1)
    x0 = x0 + U32(K0)
    x1 = x1 + U32(K1 + 3)
    x0, x1 = _rounds(x0, x1, _ROT2)
    x0 = x0 + U32(K1)
    x1 = x1 + U32(KS2 + 4)
    x0, x1 = _rounds(x0, x1, _ROT1)
    x0 = x0 + U32(KS2)
    x1 = x1 + U32(K0 + 5)
    return x0 ^ x1


def _log_noise_from_bits(bits):
    f = lax.bitcast_convert_type((bits >> U32(9)) | U32(0x3F800000), jnp.float32)
    u = jnp.maximum(f - 1.0, 0.0)
    noise = -jnp.log1p(-u)
    noise = jnp.maximum(noise, 1e-10)
    return jnp.log(noise)


def _tc_body(la_ref, aux_ref, out_ref, rv, ri):
    j = pl.program_id(0)

    @pl.when(j == 0)
    def _init():
        rv[...] = jnp.full((B, 128), -jnp.inf, jnp.float32)
        ri[...] = jnp.zeros((B, 128), jnp.int32)

    inv = aux_ref[:, :1]
    nm = aux_ref[:, 128:129]
    lane = lax.broadcasted_iota(jnp.int32, (B, 128), 1)
    rowv = lax.broadcasted_iota(jnp.uint32, (B, 128), 0) * U32(V)
    rowlane = rowv + lane.astype(jnp.uint32)

    def body(k, carry):
        rv_v, ri_v = carry
        la_k = la_ref[:, pl.ds(k * 128, 128)]
        base = j * CCH + k * 128
        ctr = rowlane + base.astype(jnp.uint32)
        ln_k = _log_noise_from_bits(_threefry_bits(ctr))
        sk = la_k * inv - ln_k * nm
        colk = lane + base
        take = (sk > rv_v) & (colk < V)
        rv_v = jnp.where(take, sk, rv_v)
        ri_v = jnp.where(take, colk, ri_v)
        return rv_v, ri_v

    rv_v, ri_v = lax.fori_loop(0, CCH // 128, body, (rv[...], ri[...]),
                               unroll=1)
    rv[...] = rv_v
    ri[...] = ri_v

    @pl.when(j == NCH - 1)
    def _finish():
        m = jnp.max(rv_v, axis=1, keepdims=True)
        ii = jnp.min(jnp.where(rv_v == m, ri_v, INTMAX), axis=1)
        out_ref[...] = ii.reshape(1, B)


@jax.jit
def _sampler(logits, aux):
    out = pl.pallas_call(
        _tc_body,
        grid=(NCH,),
        in_specs=[
            pl.BlockSpec((B, CCH), lambda j: (0, j)),
            pl.BlockSpec((B, 256), lambda j: (0, 0)),
        ],
        out_specs=pl.BlockSpec((1, B), lambda j: (0, 0)),
        out_shape=jax.ShapeDtypeStruct((1, B), jnp.int32),
        scratch_shapes=[
            pltpu.VMEM((B, 128), jnp.float32),
            pltpu.VMEM((B, 128), jnp.int32),
        ],
        compiler_params=pltpu.CompilerParams(
            dimension_semantics=("arbitrary",)),
    )(logits, aux)
    return out.reshape(B)


def kernel(logits, temperatures):
    flag = jnp.all(temperatures == 0)
    inv_t = jnp.where(flag, jnp.float32(1.0), 1.0 / temperatures)
    nmask = jnp.where(flag, jnp.float32(0.0), jnp.float32(1.0))
    aux = jnp.concatenate(
        [jnp.broadcast_to(inv_t[:, None], (B, 128)),
         jnp.broadcast_to(nmask, (B, 128))], axis=1)
    return _sampler(logits, aux)


# speedup vs baseline: 1.1264x; 1.1264x over previous
import functools

import jax
import jax.numpy as jnp
import numpy as np
from jax import lax
from jax.experimental import pallas as pl
from jax.experimental.pallas import tpu as pltpu

B = 128
V = 100000
CCH = 8192
NCH = -(-V // CCH)
INTMAX = 2**31 - 1

K0 = 0
K1 = 42
KS2 = K0 ^ K1 ^ 0x1BD11BDA
U32 = jnp.uint32

_ROT1 = (13, 15, 26, 6)
_ROT2 = (17, 29, 16, 24)


def _rounds(x0, x1, rots):
    for r in rots:
        x0 = x0 + x1
        x1 = (x1 << U32(r)) | (x1 >> U32(32 - r))
        x1 = x1 ^ x0
    return x0, x1


def _threefry_bits(ctr):
    x0 = jnp.zeros_like(ctr) + U32(K0)
    x1 = ctr + U32(K1)
    x0, x1 = _rounds(x0, x1, _ROT1)
    x0 = x0 + U32(K1)
    x1 = x1 + U32(KS2 + 1)
    x0, x1 = _rounds(x0, x1, _ROT2)
    x0 = x0 + U32(KS2)
    x1 = x1 + U32(K0 + 2)
    x0, x1 = _rounds(x0, x1, _ROT1)
    x0 = x0 + U32(K0)
    x1 = x1 + U32(K1 + 3)
    x0, x1 = _rounds(x0, x1, _ROT2)
    x0 = x0 + U32(K1)
    x1 = x1 + U32(KS2 + 4)
    x0, x1 = _rounds(x0, x1, _ROT1)
    x0 = x0 + U32(KS2)
    x1 = x1 + U32(K0 + 5)
    return x0 ^ x1


def _log_noise_from_bits(bits):
    f = lax.bitcast_convert_type((bits >> U32(9)) | U32(0x3F800000), jnp.float32)
    u = jnp.maximum(f - 1.0, 0.0)
    noise = -jnp.log1p(-u)
    noise = jnp.maximum(noise, 1e-10)
    return jnp.log(noise)


def _tc_body(la_ref, aux_ref, out_ref, rv, ri):
    j = pl.program_id(0)

    @pl.when(j == 0)
    def _init():
        rv[...] = jnp.full((B, 128), -jnp.inf, jnp.float32)
        ri[...] = jnp.zeros((B, 128), jnp.int32)

    inv = aux_ref[:, :1]
    nm = aux_ref[:, 128:129]
    lane = lax.broadcasted_iota(jnp.int32, (B, 128), 1)
    rowv = lax.broadcasted_iota(jnp.uint32, (B, 128), 0) * U32(V)
    rowlane = rowv + lane.astype(jnp.uint32)

    def body(k, carry):
        rv_v, ri_v = carry
        la_k = la_ref[:, pl.ds(k * 128, 128)]
        base = j * CCH + k * 128
        ctr = rowlane + base.astype(jnp.uint32)
        ln_k = _log_noise_from_bits(_threefry_bits(ctr))
        sk = la_k * inv - ln_k * nm
        colk = lane + base
        take = (sk > rv_v) & (colk < V)
        rv_v = jnp.where(take, sk, rv_v)
        ri_v = jnp.where(take, colk, ri_v)
        return rv_v, ri_v

    rv_v, ri_v = lax.fori_loop(0, CCH // 128, body, (rv[...], ri[...]),
                               unroll=2)
    rv[...] = rv_v
    ri[...] = ri_v

    @pl.when(j == NCH - 1)
    def _finish():
        m = jnp.max(rv_v, axis=1, keepdims=True)
        ii = jnp.min(jnp.where(rv_v == m, ri_v, INTMAX), axis=1)
        out_ref[...] = ii.reshape(1, B)


@jax.jit
def _sampler(logits, aux):
    out = pl.pallas_call(
        _tc_body,
        grid=(NCH,),
        in_specs=[
            pl.BlockSpec((B, CCH), lambda j: (0, j)),
            pl.BlockSpec((B, 256), lambda j: (0, 0)),
        ],
        out_specs=pl.BlockSpec((1, B), lambda j: (0, 0)),
        out_shape=jax.ShapeDtypeStruct((1, B), jnp.int32),
        scratch_shapes=[
            pltpu.VMEM((B, 128), jnp.float32),
            pltpu.VMEM((B, 128), jnp.int32),
        ],
        compiler_params=pltpu.CompilerParams(
            dimension_semantics=("arbitrary",)),
    )(logits, aux)
    return out.reshape(B)


def kernel(logits, temperatures):
    flag = jnp.all(temperatures == 0)
    inv_t = jnp.where(flag, jnp.float32(1.0), 1.0 / temperatures)
    nmask = jnp.where(flag, jnp.float32(0.0), jnp.float32(1.0))
    aux = jnp.concatenate(
        [jnp.broadcast_to(inv_t[:, None], (B, 128)),
         jnp.broadcast_to(nmask, (B, 128))], axis=1)
    return _sampler(logits, aux)


# speedup vs baseline: 1.1281x; 1.0015x over previous
import functools

import jax
import jax.numpy as jnp
import numpy as np
from jax import lax
from jax.experimental import pallas as pl
from jax.experimental.pallas import tpu as pltpu

B = 128
V = 100000
CCH = 8192
NCH = -(-V // CCH)
INTMAX = 2**31 - 1

K0 = 0
K1 = 42
KS2 = K0 ^ K1 ^ 0x1BD11BDA
U32 = jnp.uint32

_ROT1 = (13, 15, 26, 6)
_ROT2 = (17, 29, 16, 24)


def _rounds(x0, x1, rots):
    for r in rots:
        x0 = x0 + x1
        x1 = (x1 << U32(r)) | (x1 >> U32(32 - r))
        x1 = x1 ^ x0
    return x0, x1


def _threefry_bits(ctr):
    x0 = jnp.zeros_like(ctr) + U32(K0)
    x1 = ctr + U32(K1)
    x0, x1 = _rounds(x0, x1, _ROT1)
    x0 = x0 + U32(K1)
    x1 = x1 + U32(KS2 + 1)
    x0, x1 = _rounds(x0, x1, _ROT2)
    x0 = x0 + U32(KS2)
    x1 = x1 + U32(K0 + 2)
    x0, x1 = _rounds(x0, x1, _ROT1)
    x0 = x0 + U32(K0)
    x1 = x1 + U32(K1 + 3)
    x0, x1 = _rounds(x0, x1, _ROT2)
    x0 = x0 + U32(K1)
    x1 = x1 + U32(KS2 + 4)
    x0, x1 = _rounds(x0, x1, _ROT1)
    x0 = x0 + U32(KS2)
    x1 = x1 + U32(K0 + 5)
    return x0 ^ x1


def _log_noise_from_bits(bits):
    f = lax.bitcast_convert_type((bits >> U32(9)) | U32(0x3F800000), jnp.float32)
    u = jnp.maximum(f - 1.0, 0.0)
    noise = -jnp.log1p(-u)
    noise = jnp.maximum(noise, 1e-10)
    return jnp.log(noise)


def _tc_body(la_ref, aux_ref, out_ref, rv, ri):
    j = pl.program_id(0)

    @pl.when(j == 0)
    def _init():
        rv[...] = jnp.full((B, 128), -jnp.inf, jnp.float32)
        ri[...] = jnp.zeros((B, 128), jnp.int32)

    inv = aux_ref[:, :1]
    nm = aux_ref[:, 128:129]
    lane = lax.broadcasted_iota(jnp.int32, (B, 128), 1)
    rowv = lax.broadcasted_iota(jnp.uint32, (B, 128), 0) * U32(V)
    rowlane = rowv + lane.astype(jnp.uint32)

    def body(k, carry):
        rv_v, ri_v = carry
        la_k = la_ref[:, pl.ds(k * 128, 128)]
        base = j * CCH + k * 128
        ctr = rowlane + base.astype(jnp.uint32)
        ln_k = _log_noise_from_bits(_threefry_bits(ctr))
        sk = la_k * inv - ln_k * nm
        colk = lane + base
        take = (sk > rv_v) & (colk < V)
        rv_v = jnp.where(take, sk, rv_v)
        ri_v = jnp.where(take, colk, ri_v)
        return rv_v, ri_v

    rv_v, ri_v = lax.fori_loop(0, CCH // 128, body, (rv[...], ri[...]),
                               unroll=4)
    rv[...] = rv_v
    ri[...] = ri_v

    @pl.when(j == NCH - 1)
    def _finish():
        m = jnp.max(rv_v, axis=1, keepdims=True)
        ii = jnp.min(jnp.where(rv_v == m, ri_v, INTMAX), axis=1)
        out_ref[...] = ii.reshape(1, B)


@jax.jit
def _sampler(logits, aux):
    out = pl.pallas_call(
        _tc_body,
        grid=(NCH,),
        in_specs=[
            pl.BlockSpec((B, CCH), lambda j: (0, j)),
            pl.BlockSpec((B, 256), lambda j: (0, 0)),
        ],
        out_specs=pl.BlockSpec((1, B), lambda j: (0, 0)),
        out_shape=jax.ShapeDtypeStruct((1, B), jnp.int32),
        scratch_shapes=[
            pltpu.VMEM((B, 128), jnp.float32),
            pltpu.VMEM((B, 128), jnp.int32),
        ],
        compiler_params=pltpu.CompilerParams(
            dimension_semantics=("arbitrary",)),
    )(logits, aux)
    return out.reshape(B)


def kernel(logits, temperatures):
    flag = jnp.all(temperatures == 0)
    inv_t = jnp.where(flag, jnp.float32(1.0), 1.0 / temperatures)
    nmask = jnp.where(flag, jnp.float32(0.0), jnp.float32(1.0))
    aux = jnp.concatenate(
        [jnp.broadcast_to(inv_t[:, None], (B, 128)),
         jnp.broadcast_to(nmask, (B, 128))], axis=1)
    return _sampler(logits, aux)
